# baseline (speedup 1.0000x reference)
import jax
import jax.numpy as jnp
from jax.experimental import pallas as pl
from jax.experimental.pallas import tpu as pltpu

_TILE = 4096


def _make_kernel(n):
    def _kernel(vfull_ref, vtile_ref, gb_ref, bt_ref, c1_ref, w2_ref,
                b2_ref, w3_ref, b3_ref, w4_ref, b4_ref, out_ref, ss_ref):
        i = pl.program_id(0)

        @pl.when(i == 0)
        def _stats():
            v = vfull_ref[...]
            s = jnp.sum(v, axis=1, keepdims=True)
            ss = jnp.sum(v * v, axis=1, keepdims=True)
            mean = s / n
            var = jnp.maximum(ss / n - mean * mean, 0.0)
            scale = gb_ref[:, 0:1] * jax.lax.rsqrt(var + 1e-5)
            shift = gb_ref[:, 1:2] - mean * scale
            ss_ref[:, 0:1] = scale
            ss_ref[:, 1:2] = shift

        @pl.when(i > 0)
        def _mlp():
            v = vtile_ref[...]
            vn = v * ss_ref[:, 0:1] + ss_ref[:, 1:2]
            x = jnp.maximum(jnp.dot(bt_ref[...], vn,
                                    preferred_element_type=jnp.float32)
                            + c1_ref[...], 0.0)
            x = jnp.maximum(jnp.dot(w2_ref[...], x,
                                    preferred_element_type=jnp.float32)
                            + b2_ref[...], 0.0)
            x = jnp.maximum(jnp.dot(w3_ref[...], x,
                                    preferred_element_type=jnp.float32)
                            + b3_ref[...], 0.0)
            x4 = jnp.dot(w4_ref[...], x,
                         preferred_element_type=jnp.float32) + b4_ref[...]
            x4 = jnp.maximum(x4, 0.0)
            out_ref[...] = jnp.tanh(x4) * 0.8 + 1.0

    return _kernel


def kernel(pos, vel, bn_gamma, bn_beta, W0, b0, W1, b1, W2, b2, W3, b3,
           W4, b4):
    del pos
    n = vel.shape[0]
    tiles = -(-n // _TILE)
    npad = tiles * _TILE
    velt = jnp.pad(vel.T, ((0, 0), (0, npad - n)))

    gb = jnp.stack([bn_gamma[:2], bn_beta[:2]], axis=1)
    w1s = W1[:, :32] + W1[:, 32:]
    bt_raw = w1s @ W0[:, :2]
    c1_raw = (w1s @ (b0 + W0[:, 2] * bn_beta[2]) + b1)[:, None]

    outt = pl.pallas_call(
        _make_kernel(n),
        grid=(tiles + 1,),
        in_specs=[
            pl.BlockSpec((2, npad), lambda i: (0, 0)),
            pl.BlockSpec((2, _TILE), lambda i: (0, jnp.maximum(i - 1, 0))),
            pl.BlockSpec((2, 2), lambda i: (0, 0)),
            pl.BlockSpec((64, 2), lambda i: (0, 0)),
            pl.BlockSpec((64, 1), lambda i: (0, 0)),
            pl.BlockSpec((128, 64), lambda i: (0, 0)),
            pl.BlockSpec((128, 1), lambda i: (0, 0)),
            pl.BlockSpec((256, 128), lambda i: (0, 0)),
            pl.BlockSpec((256, 1), lambda i: (0, 0)),
            pl.BlockSpec((1, 256), lambda i: (0, 0)),
            pl.BlockSpec((1, 1), lambda i: (0, 0)),
        ],
        out_specs=pl.BlockSpec((1, _TILE), lambda i: (0, jnp.maximum(i - 1, 0))),
        out_shape=jax.ShapeDtypeStruct((1, n), jnp.float32),
        scratch_shapes=[pltpu.VMEM((2, 2), jnp.float32)],
        compiler_params=pltpu.CompilerParams(
            dimension_semantics=("arbitrary",)),
    )(velt, velt, gb, bt_raw, c1_raw, W2, b2[:, None], W3, b3[:, None],
      W4, b4[:, None])

    return outt.T


# speedup vs baseline: 1.0470x; 1.0470x over previous
import jax
import jax.numpy as jnp
from jax.experimental import pallas as pl
from jax.experimental.pallas import tpu as pltpu

_TILE = 8192


def _make_kernel(n):
    def _kernel(vfull_ref, vtile_ref, gb_ref, btr_ref, c1r_ref, w2_ref,
                b2_ref, w3_ref, b3_ref, w4_ref, b4_ref, out_ref, bt_ref,
                c1_ref):
        i = pl.program_id(0)

        @pl.when(i == 0)
        def _stats():
            v = vfull_ref[...]
            s = jnp.sum(v, axis=1, keepdims=True)
            ss = jnp.sum(v * v, axis=1, keepdims=True)
            mean = s / n
            var = jnp.maximum(ss / n - mean * mean, 0.0)
            scale = gb_ref[:, 0:1] * jax.lax.rsqrt(var + 1e-5)
            shift = gb_ref[:, 1:2] - mean * scale
            bt_ref[...] = btr_ref[...] * scale.T
            c1_ref[...] = c1r_ref[...] + jnp.dot(
                btr_ref[...], shift, preferred_element_type=jnp.float32)

        @pl.when(i > 0)
        def _mlp():
            v = vtile_ref[...]
            x = jnp.maximum(jnp.dot(bt_ref[...], v,
                                    preferred_element_type=jnp.float32)
                            + c1_ref[...], 0.0)
            x = jnp.maximum(jnp.dot(w2_ref[...], x,
                                    preferred_element_type=jnp.float32)
                            + b2_ref[...], 0.0)
            x = jnp.maximum(jnp.dot(w3_ref[...], x,
                                    preferred_element_type=jnp.float32)
                            + b3_ref[...], 0.0)
            x4 = jnp.dot(w4_ref[...], x,
                         preferred_element_type=jnp.float32) + b4_ref[...]
            x4 = jnp.maximum(x4, 0.0)
            out_ref[...] = jnp.tanh(x4) * 0.8 + 1.0

    return _kernel


def kernel(pos, vel, bn_gamma, bn_beta, W0, b0, W1, b1, W2, b2, W3, b3,
           W4, b4):
    del pos
    n = vel.shape[0]
    tiles = -(-n // _TILE)
    npad = tiles * _TILE
    velt = jnp.pad(vel.T, ((0, 0), (0, npad - n)))

    gb = jnp.stack([bn_gamma[:2], bn_beta[:2]], axis=1)
    w1s = W1[:, :32] + W1[:, 32:]
    bt_raw = w1s @ W0[:, :2]
    c1_raw = (w1s @ (b0 + W0[:, 2] * bn_beta[2]) + b1)[:, None]

    outt = pl.pallas_call(
        _make_kernel(n),
        grid=(tiles + 1,),
        in_specs=[
            pl.BlockSpec((2, npad), lambda i: (0, 0)),
            pl.BlockSpec((2, _TILE), lambda i: (0, jnp.maximum(i - 1, 0))),
            pl.BlockSpec((2, 2), lambda i: (0, 0)),
            pl.BlockSpec((64, 2), lambda i: (0, 0)),
            pl.BlockSpec((64, 1), lambda i: (0, 0)),
            pl.BlockSpec((128, 64), lambda i: (0, 0)),
            pl.BlockSpec((128, 1), lambda i: (0, 0)),
            pl.BlockSpec((256, 128), lambda i: (0, 0)),
            pl.BlockSpec((256, 1), lambda i: (0, 0)),
            pl.BlockSpec((1, 256), lambda i: (0, 0)),
            pl.BlockSpec((1, 1), lambda i: (0, 0)),
        ],
        out_specs=pl.BlockSpec((1, _TILE), lambda i: (0, jnp.maximum(i - 1, 0))),
        out_shape=jax.ShapeDtypeStruct((1, n), jnp.float32),
        scratch_shapes=[pltpu.VMEM((64, 2), jnp.float32),
                        pltpu.VMEM((64, 1), jnp.float32)],
        compiler_params=pltpu.CompilerParams(
            dimension_semantics=("arbitrary",)),
    )(velt, velt, gb, bt_raw, c1_raw, W2, b2[:, None],
      W3, b3[:, None], W4, b4[:, None])

    return outt.T


# speedup vs baseline: 1.0474x; 1.0004x over previous
import jax
import jax.numpy as jnp
from jax.experimental import pallas as pl
from jax.experimental.pallas import tpu as pltpu

_TILE = 8192


def _make_kernel(n):
    def _kernel(vfull_ref, gb_ref, btr_ref, c1r_ref, w2_ref,
                b2_ref, w3_ref, b3_ref, w4_ref, b4_ref, out_ref, bt_ref,
                c1_ref):
        i = pl.program_id(0)

        @pl.when(i == 0)
        def _stats():
            v = vfull_ref[...]
            s = jnp.sum(v, axis=1, keepdims=True)
            ss = jnp.sum(v * v, axis=1, keepdims=True)
            mean = s / n
            var = jnp.maximum(ss / n - mean * mean, 0.0)
            scale = gb_ref[:, 0:1] * jax.lax.rsqrt(var + 1e-5)
            shift = gb_ref[:, 1:2] - mean * scale
            bt_ref[...] = btr_ref[...] * scale.T
            c1_ref[...] = c1r_ref[...] + jnp.dot(
                btr_ref[...], shift, preferred_element_type=jnp.float32)

        @pl.when(i > 0)
        def _mlp():
            v = vfull_ref[:, pl.ds((i - 1) * _TILE, _TILE)]
            x = jnp.maximum(jnp.dot(bt_ref[...], v,
                                    preferred_element_type=jnp.float32)
                            + c1_ref[...], 0.0)
            x = jnp.maximum(jnp.dot(w2_ref[...], x,
                                    preferred_element_type=jnp.float32)
                            + b2_ref[...], 0.0)
            x = jnp.maximum(jnp.dot(w3_ref[...], x,
                                    preferred_element_type=jnp.float32)
                            + b3_ref[...], 0.0)
            x4 = jnp.dot(w4_ref[...], x,
                         preferred_element_type=jnp.float32) + b4_ref[...]
            x4 = jnp.maximum(x4, 0.0)
            out_ref[...] = jnp.tanh(x4) * 0.8 + 1.0

    return _kernel


def kernel(pos, vel, bn_gamma, bn_beta, W0, b0, W1, b1, W2, b2, W3, b3,
           W4, b4):
    del pos
    n = vel.shape[0]
    tiles = -(-n // _TILE)
    npad = tiles * _TILE
    velt = jnp.pad(vel.T, ((0, 0), (0, npad - n)))

    gb = jnp.stack([bn_gamma[:2], bn_beta[:2]], axis=1)
    w1s = W1[:, :32] + W1[:, 32:]
    bt_raw = w1s @ W0[:, :2]
    c1_raw = (w1s @ (b0 + W0[:, 2] * bn_beta[2]) + b1)[:, None]

    outt = pl.pallas_call(
        _make_kernel(n),
        grid=(tiles + 1,),
        in_specs=[
            pl.BlockSpec((2, npad), lambda i: (0, 0)),
            pl.BlockSpec((2, 2), lambda i: (0, 0)),
            pl.BlockSpec((64, 2), lambda i: (0, 0)),
            pl.BlockSpec((64, 1), lambda i: (0, 0)),
            pl.BlockSpec((128, 64), lambda i: (0, 0)),
            pl.BlockSpec((128, 1), lambda i: (0, 0)),
            pl.BlockSpec((256, 128), lambda i: (0, 0)),
            pl.BlockSpec((256, 1), lambda i: (0, 0)),
            pl.BlockSpec((1, 256), lambda i: (0, 0)),
            pl.BlockSpec((1, 1), lambda i: (0, 0)),
        ],
        out_specs=pl.BlockSpec((1, _TILE), lambda i: (0, jnp.maximum(i - 1, 0))),
        out_shape=jax.ShapeDtypeStruct((1, n), jnp.float32),
        scratch_shapes=[pltpu.VMEM((64, 2), jnp.float32),
                        pltpu.VMEM((64, 1), jnp.float32)],
        compiler_params=pltpu.CompilerParams(
            dimension_semantics=("arbitrary",)),
    )(velt, gb, bt_raw, c1_raw, W2, b2[:, None],
      W3, b3[:, None], W4, b4[:, None])

    return outt.T


# speedup vs baseline: 1.0850x; 1.0359x over previous
import jax
import jax.numpy as jnp
from jax.experimental import pallas as pl
from jax.experimental.pallas import tpu as pltpu

_TILE = 10240


def _make_kernel(n):
    def _kernel(vfull_ref, gb_ref, btr_ref, c1r_ref, w2_ref,
                b2_ref, w3_ref, b3_ref, w4_ref, b4_ref, out_ref, bt_ref,
                c1_ref):
        i = pl.program_id(0)

        @pl.when(i == 0)
        def _stats():
            v = vfull_ref[...]
            s = jnp.sum(v, axis=1, keepdims=True)
            ss = jnp.sum(v * v, axis=1, keepdims=True)
            mean = s / n
            var = jnp.maximum(ss / n - mean * mean, 0.0)
            scale = gb_ref[:, 0:1] * jax.lax.rsqrt(var + 1e-5)
            shift = gb_ref[:, 1:2] - mean * scale
            bt_ref[...] = btr_ref[...] * scale.T
            c1_ref[...] = c1r_ref[...] + jnp.dot(
                btr_ref[...], shift, preferred_element_type=jnp.float32)

        @pl.when(i > 0)
        def _mlp():
            v = vfull_ref[:, pl.ds((i - 1) * _TILE, _TILE)]
            x = jnp.maximum(jnp.dot(bt_ref[...], v,
                                    preferred_element_type=jnp.float32)
                            + c1_ref[...], 0.0)
            x = jnp.maximum(jnp.dot(w2_ref[...], x,
                                    preferred_element_type=jnp.float32)
                            + b2_ref[...], 0.0)
            x = jnp.maximum(jnp.dot(w3_ref[...], x,
                                    preferred_element_type=jnp.float32)
                            + b3_ref[...], 0.0)
            x4 = jnp.dot(w4_ref[...], x,
                         preferred_element_type=jnp.float32) + b4_ref[...]
            x4 = jnp.maximum(x4, 0.0)
            out_ref[...] = jnp.tanh(x4) * 0.8 + 1.0

    return _kernel


def kernel(pos, vel, bn_gamma, bn_beta, W0, b0, W1, b1, W2, b2, W3, b3,
           W4, b4):
    del pos
    n = vel.shape[0]
    tiles = -(-n // _TILE)
    npad = tiles * _TILE
    velt = jnp.pad(vel.T, ((0, 0), (0, npad - n)))

    gb = jnp.stack([bn_gamma[:2], bn_beta[:2]], axis=1)
    w1s = W1[:, :32] + W1[:, 32:]
    bt_raw = w1s @ W0[:, :2]
    c1_raw = (w1s @ (b0 + W0[:, 2] * bn_beta[2]) + b1)[:, None]

    outt = pl.pallas_call(
        _make_kernel(n),
        grid=(tiles + 1,),
        in_specs=[
            pl.BlockSpec((2, npad), lambda i: (0, 0)),
            pl.BlockSpec((2, 2), lambda i: (0, 0)),
            pl.BlockSpec((64, 2), lambda i: (0, 0)),
            pl.BlockSpec((64, 1), lambda i: (0, 0)),
            pl.BlockSpec((128, 64), lambda i: (0, 0)),
            pl.BlockSpec((128, 1), lambda i: (0, 0)),
            pl.BlockSpec((256, 128), lambda i: (0, 0)),
            pl.BlockSpec((256, 1), lambda i: (0, 0)),
            pl.BlockSpec((1, 256), lambda i: (0, 0)),
            pl.BlockSpec((1, 1), lambda i: (0, 0)),
        ],
        out_specs=pl.BlockSpec((1, _TILE), lambda i: (0, jnp.maximum(i - 1, 0))),
        out_shape=jax.ShapeDtypeStruct((1, n), jnp.float32),
        scratch_shapes=[pltpu.VMEM((64, 2), jnp.float32),
                        pltpu.VMEM((64, 1), jnp.float32)],
        compiler_params=pltpu.CompilerParams(
            dimension_semantics=("arbitrary",)),
    )(velt, gb, bt_raw, c1_raw, W2, b2[:, None],
      W3, b3[:, None], W4, b4[:, None])

    return outt.T


# speedup vs baseline: 1.0959x; 1.0100x over previous
import jax
import jax.numpy as jnp
from jax.experimental import pallas as pl
from jax.experimental.pallas import tpu as pltpu

_TILE = 12800


def _make_kernel(n):
    def _kernel(vfull_ref, gb_ref, btr_ref, c1r_ref, w2_ref,
                b2_ref, w3_ref, b3_ref, w4_ref, b4_ref, out_ref, bt_ref,
                c1_ref):
        i = pl.program_id(0)

        @pl.when(i == 0)
        def _stats():
            v = vfull_ref[...]
            s = jnp.sum(v, axis=1, keepdims=True)
            ss = jnp.sum(v * v, axis=1, keepdims=True)
            mean = s / n
            var = jnp.maximum(ss / n - mean * mean, 0.0)
            scale = gb_ref[:, 0:1] * jax.lax.rsqrt(var + 1e-5)
            shift = gb_ref[:, 1:2] - mean * scale
            bt_ref[...] = btr_ref[...] * scale.T
            c1_ref[...] = c1r_ref[...] + jnp.dot(
                btr_ref[...], shift, preferred_element_type=jnp.float32)

        @pl.when(i > 0)
        def _mlp():
            v = vfull_ref[:, pl.ds((i - 1) * _TILE, _TILE)]
            x = jnp.maximum(jnp.dot(bt_ref[...], v,
                                    preferred_element_type=jnp.float32)
                            + c1_ref[...], 0.0)
            x = jnp.maximum(jnp.dot(w2_ref[...], x,
                                    preferred_element_type=jnp.float32)
                            + b2_ref[...], 0.0)
            x = jnp.maximum(jnp.dot(w3_ref[...], x,
                                    preferred_element_type=jnp.float32)
                            + b3_ref[...], 0.0)
            x4 = jnp.dot(w4_ref[...], x,
                         preferred_element_type=jnp.float32) + b4_ref[...]
            x4 = jnp.maximum(x4, 0.0)
            out_ref[...] = jnp.tanh(x4) * 0.8 + 1.0

    return _kernel


def kernel(pos, vel, bn_gamma, bn_beta, W0, b0, W1, b1, W2, b2, W3, b3,
           W4, b4):
    del pos
    n = vel.shape[0]
    tiles = -(-n // _TILE)
    npad = tiles * _TILE
    velt = jnp.pad(vel.T, ((0, 0), (0, npad - n)))

    gb = jnp.stack([bn_gamma[:2], bn_beta[:2]], axis=1)
    w1s = W1[:, :32] + W1[:, 32:]
    bt_raw = w1s @ W0[:, :2]
    c1_raw = (w1s @ (b0 + W0[:, 2] * bn_beta[2]) + b1)[:, None]

    outt = pl.pallas_call(
        _make_kernel(n),
        grid=(tiles + 1,),
        in_specs=[
            pl.BlockSpec((2, npad), lambda i: (0, 0)),
            pl.BlockSpec((2, 2), lambda i: (0, 0)),
            pl.BlockSpec((64, 2), lambda i: (0, 0)),
            pl.BlockSpec((64, 1), lambda i: (0, 0)),
            pl.BlockSpec((128, 64), lambda i: (0, 0)),
            pl.BlockSpec((128, 1), lambda i: (0, 0)),
            pl.BlockSpec((256, 128), lambda i: (0, 0)),
            pl.BlockSpec((256, 1), lambda i: (0, 0)),
            pl.BlockSpec((1, 256), lambda i: (0, 0)),
            pl.BlockSpec((1, 1), lambda i: (0, 0)),
        ],
        out_specs=pl.BlockSpec((1, _TILE), lambda i: (0, jnp.maximum(i - 1, 0))),
        out_shape=jax.ShapeDtypeStruct((1, n), jnp.float32),
        scratch_shapes=[pltpu.VMEM((64, 2), jnp.float32),
                        pltpu.VMEM((64, 1), jnp.float32)],
        compiler_params=pltpu.CompilerParams(
            dimension_semantics=("arbitrary",)),
    )(velt, gb, bt_raw, c1_raw, W2, b2[:, None],
      W3, b3[:, None], W4, b4[:, None])

    return outt.T


# speedup vs baseline: 1.1110x; 1.0138x over previous
import jax
import jax.numpy as jnp
from jax.experimental import pallas as pl
from jax.experimental.pallas import tpu as pltpu

_TILE = 20480


def _make_kernel(n):
    def _kernel(vfull_ref, gb_ref, btr_ref, c1r_ref, w2_ref,
                b2_ref, w3_ref, b3_ref, w4_ref, b4_ref, out_ref, bt_ref,
                c1_ref):
        i = pl.program_id(0)

        @pl.when(i == 0)
        def _stats():
            v = vfull_ref[...]
            s = jnp.sum(v, axis=1, keepdims=True)
            ss = jnp.sum(v * v, axis=1, keepdims=True)
            mean = s / n
            var = jnp.maximum(ss / n - mean * mean, 0.0)
            scale = gb_ref[:, 0:1] * jax.lax.rsqrt(var + 1e-5)
            shift = gb_ref[:, 1:2] - mean * scale
            bt_ref[...] = btr_ref[...] * scale.T
            c1_ref[...] = c1r_ref[...] + jnp.dot(
                btr_ref[...], shift, preferred_element_type=jnp.float32)

        @pl.when(i > 0)
        def _mlp():
            v = vfull_ref[:, pl.ds((i - 1) * _TILE, _TILE)]
            x = jnp.maximum(jnp.dot(bt_ref[...], v,
                                    preferred_element_type=jnp.float32)
                            + c1_ref[...], 0.0)
            x = jnp.maximum(jnp.dot(w2_ref[...], x,
                                    preferred_element_type=jnp.float32)
                            + b2_ref[...], 0.0)
            x = jnp.maximum(jnp.dot(w3_ref[...], x,
                                    preferred_element_type=jnp.float32)
                            + b3_ref[...], 0.0)
            x4 = jnp.dot(w4_ref[...], x,
                         preferred_element_type=jnp.float32) + b4_ref[...]
            x4 = jnp.maximum(x4, 0.0)
            out_ref[...] = jnp.tanh(x4) * 0.8 + 1.0

    return _kernel


def kernel(pos, vel, bn_gamma, bn_beta, W0, b0, W1, b1, W2, b2, W3, b3,
           W4, b4):
    del pos
    n = vel.shape[0]
    tiles = -(-n // _TILE)
    npad = tiles * _TILE
    velt = jnp.pad(vel.T, ((0, 0), (0, npad - n)))

    gb = jnp.stack([bn_gamma[:2], bn_beta[:2]], axis=1)
    w1s = W1[:, :32] + W1[:, 32:]
    bt_raw = w1s @ W0[:, :2]
    c1_raw = (w1s @ (b0 + W0[:, 2] * bn_beta[2]) + b1)[:, None]

    outt = pl.pallas_call(
        _make_kernel(n),
        grid=(tiles + 1,),
        in_specs=[
            pl.BlockSpec((2, npad), lambda i: (0, 0)),
            pl.BlockSpec((2, 2), lambda i: (0, 0)),
            pl.BlockSpec((64, 2), lambda i: (0, 0)),
            pl.BlockSpec((64, 1), lambda i: (0, 0)),
            pl.BlockSpec((128, 64), lambda i: (0, 0)),
            pl.BlockSpec((128, 1), lambda i: (0, 0)),
            pl.BlockSpec((256, 128), lambda i: (0, 0)),
            pl.BlockSpec((256, 1), lambda i: (0, 0)),
            pl.BlockSpec((1, 256), lambda i: (0, 0)),
            pl.BlockSpec((1, 1), lambda i: (0, 0)),
        ],
        out_specs=pl.BlockSpec((1, _TILE), lambda i: (0, jnp.maximum(i - 1, 0))),
        out_shape=jax.ShapeDtypeStruct((1, n), jnp.float32),
        scratch_shapes=[pltpu.VMEM((64, 2), jnp.float32),
                        pltpu.VMEM((64, 1), jnp.float32)],
        compiler_params=pltpu.CompilerParams(
            dimension_semantics=("arbitrary",)),
    )(velt, gb, bt_raw, c1_raw, W2, b2[:, None],
      W3, b3[:, None], W4, b4[:, None])

    return outt.T
